# baseline (speedup 1.0000x reference)
import functools

import jax
import jax.numpy as jnp
from jax import lax
from jax.experimental import pallas as pl
from jax.experimental.pallas import tpu as pltpu
from jax.experimental.pallas import tpu_sc as plsc

N = 10000
E = 320000
D = 128
H = 1
F = 128

NC = 2
NS = 16

CH = 80
EB = E // (NC * NS)
BG = 2000
DEPTH = 5
NROW = 4
TOUT = 624
WOUT = 640

BN = 1000


def _tc_proj_body(x_ref, w_ref, al_ref, ar_ref,
                  feat_ref, el_ref, er_ref, gm_ref):
    i = pl.program_id(0)
    xb = x_ref[...]
    w = w_ref[...]
    feat = lax.dot_general(xb, w, (((1,), (1,)), ((), ())),
                           preferred_element_type=jnp.float32)
    feat_ref[...] = feat
    el = jnp.sum(feat * al_ref[...], axis=1, keepdims=True)
    el_ref[...] = el
    er_ref[...] = jnp.sum(feat * ar_ref[...], axis=1, keepdims=True)

    @pl.when(i == 0)
    def _():
        gm_ref[...] = jnp.full((1, F), -jnp.inf, jnp.float32)
    gm_ref[...] = jnp.maximum(gm_ref[...], jnp.max(el))


def _tc_proj(x, W, al, ar):
    return pl.pallas_call(
        _tc_proj_body,
        grid=(N // BN,),
        in_specs=[
            pl.BlockSpec((BN, D), lambda i: (i, 0)),
            pl.BlockSpec((D, D), lambda i: (0, 0)),
            pl.BlockSpec((1, F), lambda i: (0, 0)),
            pl.BlockSpec((1, F), lambda i: (0, 0)),
        ],
        out_specs=[
            pl.BlockSpec((BN, F), lambda i: (i, 0)),
            pl.BlockSpec((BN, 1), lambda i: (i, 0)),
            pl.BlockSpec((BN, 1), lambda i: (i, 0)),
            pl.BlockSpec((1, F), lambda i: (0, 0)),
        ],
        out_shape=[
            jax.ShapeDtypeStruct((N, F), jnp.float32),
            jax.ShapeDtypeStruct((N, 1), jnp.float32),
            jax.ShapeDtypeStruct((N, 1), jnp.float32),
            jax.ShapeDtypeStruct((1, F), jnp.float32),
        ],
    )(x, W, al, ar)


def _tc_combine_body(p_ref, d_ref, x_ref, b_ref, o_ref):
    p = p_ref[...]
    d = d_ref[...]
    den = d[0] + d[1] + jnp.float32(1e-9)
    o_ref[...] = (p[0] + p[1]) / den + x_ref[...] + b_ref[...]


def _tc_combine(parts, dens, x, bias):
    return pl.pallas_call(
        _tc_combine_body,
        grid=(N // BN,),
        in_specs=[
            pl.BlockSpec((2, BN, F), lambda i: (0, i, 0)),
            pl.BlockSpec((2, BN, 1), lambda i: (0, i, 0)),
            pl.BlockSpec((BN, D), lambda i: (i, 0)),
            pl.BlockSpec((1, F), lambda i: (0, 0)),
        ],
        out_specs=pl.BlockSpec((BN, F), lambda i: (i, 0)),
        out_shape=jax.ShapeDtypeStruct((N, F), jnp.float32),
    )(parts, dens, x, bias)


def _leaky(v):
    return jnp.where(v > 0, v, jnp.float32(0.2) * v)


def _sc_edge_body(src_hbm, dst_hbm, el_hbm, er_hbm, gmax_hbm, feat_hbm,
                  acc_hbm, den_hbm,
                  gmax_t, src_st, dst_st,
                  src_sc, dst_sc, w_buf, el_c, er_c, rows_v, den_o,
                  el_s, er_s, den_s, acc_s,
                  gsem, ssem, dsem, esem):
    t = lax.axis_index("s")
    c = lax.axis_index("c")
    wid = c * NS + t

    pltpu.sync_copy(gmax_hbm, gmax_t)

    @pl.when(t == 0)
    def _():
        pltpu.sync_copy(el_hbm, el_s)
        pltpu.sync_copy(er_hbm, er_s)

    zv = jnp.zeros((16,), jnp.float32)

    for v in range(5):
        el_c[0][pl.ds(v * 16, 16)] = zv

    @pl.when(t < 5)
    def _():
        def _zden(i, carry):
            pltpu.sync_copy(el_c[0],
                            den_s.at[pl.ds(t * 2000 + i * CH, CH)])
            return carry
        lax.fori_loop(0, 2000 // CH, _zden, 0)

    def _zrows(i, carry):
        for k in range(8):
            rows_v[0][i, pl.ds(k * 16, 16)] = zv
        return carry
    lax.fori_loop(0, CH, _zrows, 0)

    def _zacc(k, carry):
        pltpu.sync_copy(rows_v[0], acc_s.at[pl.ds(t * TOUT + k * CH, CH)])
        return carry
    lax.fori_loop(0, WOUT // CH, _zacc, 0)

    gmax = gmax_t[...]

    plsc.subcore_barrier()

    def _group(g, carry0):
        base = wid * EB + g * BG
        pltpu.sync_copy(src_hbm.at[pl.ds(base, BG)], src_st)
        pltpu.sync_copy(dst_hbm.at[pl.ds(base, BG)], dst_st)

        def _body(ib, carry):
            for k in range(DEPTH):
                i = ib * DEPTH + k
                for v in range(5):
                    src_sc[k][pl.ds(v * 16, 16)] = (
                        src_st[pl.ds(i * CH + v * 16, 16)])
                    dst_sc[k][pl.ds(v * 16, 16)] = (
                        dst_st[pl.ds(i * CH + v * 16, 16)])
            gds = [None] * DEPTH
            for k in range(NROW):
                gds[k] = pltpu.async_copy(
                    feat_hbm.at[src_sc[k]], rows_v[k], gsem[k])
            eds = []
            for k in range(DEPTH):
                eds.append((
                    pltpu.async_copy(el_s.at[src_sc[k]], el_c[k],
                                     esem[2 * k]),
                    pltpu.async_copy(er_s.at[dst_sc[k]], er_c[k],
                                     esem[2 * k + 1]),
                ))

            sds = [None] * DEPTH
            dds = []
            for k in range(DEPTH):
                eds[k][0].wait()
                eds[k][1].wait()
                for v in range(5):
                    elv = el_c[k][pl.ds(v * 16, 16)]
                    erv = er_c[k][pl.ds(v * 16, 16)]
                    e = _leaky(elv + erv)
                    cv = _leaky(gmax + erv)
                    w_buf[k][pl.ds(v * 16, 16)] = jnp.exp(e - cv)
                dds.append(pltpu.async_copy(
                    w_buf[k], den_s.at[dst_sc[k]], dsem[k], add=True))

                gds[k].wait()
                r = rows_v[k % NROW]

                def _scale(j, carry2, r=r, k=k):
                    aj = plsc.load_gather(
                        w_buf[k], [jnp.full((16,), j, jnp.int32)])
                    for f8 in range(8):
                        r[j, pl.ds(f8 * 16, 16)] = (
                            r[j, pl.ds(f8 * 16, 16)] * aj)
                    return carry2
                lax.fori_loop(0, CH, _scale, 0, unroll=8)

                sds[k] = pltpu.async_copy(
                    r, acc_s.at[dst_sc[k]], ssem[k], add=True)
                if k == DEPTH - NROW:
                    sds[0].wait()
                    gds[DEPTH - 1] = pltpu.async_copy(
                        feat_hbm.at[src_sc[DEPTH - 1]],
                        rows_v[0], gsem[0])
            for k in range(1, DEPTH):
                sds[k].wait()
            for d in dds:
                d.wait()
            return carry
        lax.fori_loop(0, BG // CH // DEPTH, _body, 0)
        return carry0
    lax.fori_loop(0, EB // BG, _group, 0)

    plsc.subcore_barrier()
    pltpu.sync_copy(acc_s.at[pl.ds(t * TOUT, WOUT)],
                    acc_hbm.at[pl.ds(c * N + t * TOUT, WOUT)])
    pltpu.sync_copy(den_s.at[pl.ds(t * TOUT, WOUT)], den_o)
    pltpu.sync_copy(den_o, den_hbm.at[pl.ds(c * N + t * TOUT, WOUT)])


def _sc_edge_wrap(src_hbm, dst_hbm, el_hbm, er_hbm, gmax_hbm, feat_hbm,
                  acc_hbm, den_hbm, gmax_t, src_st, dst_st, *rest):
    src_sc = list(rest[0:DEPTH])
    dst_sc = list(rest[DEPTH:2 * DEPTH])
    w_buf = list(rest[2 * DEPTH:3 * DEPTH])
    el_c = list(rest[3 * DEPTH:4 * DEPTH])
    er_c = list(rest[4 * DEPTH:5 * DEPTH])
    rows_v = list(rest[5 * DEPTH:5 * DEPTH + NROW])
    o = 5 * DEPTH + NROW
    den_o = rest[o]
    o += 1
    el_s, er_s, den_s, acc_s = rest[o:o + 4]
    gsem = list(rest[o + 4:o + 4 + NROW])
    ssem = list(rest[o + 4 + NROW:o + 4 + NROW + DEPTH])
    b = o + 4 + NROW + DEPTH
    dsem = list(rest[b:b + DEPTH])
    esem = list(rest[b + DEPTH:b + 3 * DEPTH])
    _sc_edge_body(src_hbm, dst_hbm, el_hbm, er_hbm, gmax_hbm, feat_hbm,
                  acc_hbm, den_hbm, gmax_t, src_st, dst_st,
                  src_sc, dst_sc, w_buf, el_c, er_c, rows_v, den_o,
                  el_s, er_s, den_s, acc_s, gsem, ssem, dsem, esem)


_sc_edge = functools.partial(
    pl.kernel,
    out_type=(
        jax.ShapeDtypeStruct((NC * N, F), jnp.float32),
        jax.ShapeDtypeStruct((NC * N,), jnp.float32),
    ),
    mesh=plsc.VectorSubcoreMesh(core_axis_name="c", subcore_axis_name="s"),
    compiler_params=pltpu.CompilerParams(needs_layout_passes=False),
    scratch_types=[
        pltpu.VMEM((16,), jnp.float32),
        pltpu.VMEM((BG,), jnp.int32),
        pltpu.VMEM((BG,), jnp.int32),
        *[pltpu.VMEM((CH,), jnp.int32) for _ in range(DEPTH)],
        *[pltpu.VMEM((CH,), jnp.int32) for _ in range(DEPTH)],
        *[pltpu.VMEM((CH,), jnp.float32) for _ in range(DEPTH)],
        *[pltpu.VMEM((CH,), jnp.float32) for _ in range(DEPTH)],
        *[pltpu.VMEM((CH,), jnp.float32) for _ in range(DEPTH)],
        *[pltpu.VMEM((CH, F), jnp.float32) for _ in range(NROW)],
        pltpu.VMEM((WOUT,), jnp.float32),
        pltpu.VMEM_SHARED((N,), jnp.float32),
        pltpu.VMEM_SHARED((N,), jnp.float32),
        pltpu.VMEM_SHARED((N,), jnp.float32),
        pltpu.VMEM_SHARED((N, F), jnp.float32),
        *[pltpu.SemaphoreType.DMA for _ in range(NROW)],
        *[pltpu.SemaphoreType.DMA for _ in range(DEPTH)],
        *[pltpu.SemaphoreType.DMA for _ in range(DEPTH)],
        *[pltpu.SemaphoreType.DMA for _ in range(2 * DEPTH)],
    ],
)(_sc_edge_wrap)


def kernel(x, edge_index, W, attn_l, attn_r, bias):
    src = edge_index[0]
    dst = edge_index[1]

    feat, el2, er2, gm = _tc_proj(x, W,
                                  attn_l.reshape(1, F), attn_r.reshape(1, F))
    el = el2.reshape(N)
    er = er2.reshape(N)
    gm16 = gm.reshape(F)[:16]

    acc, den = _sc_edge(src, dst, el, er, gm16, feat)
    parts = acc.reshape(NC, N, F)
    dens = den.reshape(NC, N, 1)

    out = _tc_combine(parts, dens, x, bias.reshape(1, F))
    return out.reshape(N, H, F)


# speedup vs baseline: 1.0309x; 1.0309x over previous
import functools

import jax
import jax.numpy as jnp
from jax import lax
from jax.experimental import pallas as pl
from jax.experimental.pallas import tpu as pltpu
from jax.experimental.pallas import tpu_sc as plsc

N = 10000
E = 320000
D = 128
H = 1
F = 128

NC = 2
NS = 16

CH = 80
EB = E // (NC * NS)
BG = 2000
DEPTH = 5
NROW = 4
TOUT = 624
WOUT = 640

BN = 1000


def _tc_proj_body(x_ref, w_ref, al_ref, ar_ref,
                  feat_ref, el_ref, er_ref, gm_ref):
    i = pl.program_id(0)
    xb = x_ref[...]
    w = w_ref[...]
    feat = lax.dot_general(xb, w, (((1,), (1,)), ((), ())),
                           preferred_element_type=jnp.float32)
    feat_ref[...] = feat
    el = jnp.sum(feat * al_ref[...], axis=1, keepdims=True)
    el_ref[...] = el
    er_ref[...] = jnp.sum(feat * ar_ref[...], axis=1, keepdims=True)

    @pl.when(i == 0)
    def _():
        gm_ref[...] = jnp.full((1, F), -jnp.inf, jnp.float32)
    gm_ref[...] = jnp.maximum(gm_ref[...], jnp.max(el))


def _tc_proj(x, W, al, ar):
    return pl.pallas_call(
        _tc_proj_body,
        grid=(N // BN,),
        in_specs=[
            pl.BlockSpec((BN, D), lambda i: (i, 0)),
            pl.BlockSpec((D, D), lambda i: (0, 0)),
            pl.BlockSpec((1, F), lambda i: (0, 0)),
            pl.BlockSpec((1, F), lambda i: (0, 0)),
        ],
        out_specs=[
            pl.BlockSpec((BN, F), lambda i: (i, 0)),
            pl.BlockSpec((BN, 1), lambda i: (i, 0)),
            pl.BlockSpec((BN, 1), lambda i: (i, 0)),
            pl.BlockSpec((1, F), lambda i: (0, 0)),
        ],
        out_shape=[
            jax.ShapeDtypeStruct((N, F), jnp.float32),
            jax.ShapeDtypeStruct((N, 1), jnp.float32),
            jax.ShapeDtypeStruct((N, 1), jnp.float32),
            jax.ShapeDtypeStruct((1, F), jnp.float32),
        ],
    )(x, W, al, ar)


def _tc_combine_body(p_ref, d_ref, x_ref, b_ref, o_ref):
    p = p_ref[...]
    d = d_ref[...]
    den = d[0] + d[1] + jnp.float32(1e-9)
    o_ref[...] = (p[0] + p[1]) / den + x_ref[...] + b_ref[...]


def _tc_combine(parts, dens, x, bias):
    return pl.pallas_call(
        _tc_combine_body,
        grid=(N // BN,),
        in_specs=[
            pl.BlockSpec((2, BN, F), lambda i: (0, i, 0)),
            pl.BlockSpec((2, BN, 1), lambda i: (0, i, 0)),
            pl.BlockSpec((BN, D), lambda i: (i, 0)),
            pl.BlockSpec((1, F), lambda i: (0, 0)),
        ],
        out_specs=pl.BlockSpec((BN, F), lambda i: (i, 0)),
        out_shape=jax.ShapeDtypeStruct((N, F), jnp.float32),
    )(parts, dens, x, bias)


def _leaky(v):
    return jnp.where(v > 0, v, jnp.float32(0.2) * v)


def _sc_edge_body(src_hbm, dst_hbm, el_hbm, er_hbm, gmax_hbm, feat_hbm,
                  acc_hbm, den_hbm,
                  gmax_t, src_st, dst_st,
                  src_sc, dst_sc, w_buf, el_c, er_c, rows_v, den_o,
                  el_s, er_s, den_s, acc_s,
                  gsem, ssem, dsem, esem):
    t = lax.axis_index("s")
    c = lax.axis_index("c")
    wid = c * NS + t

    pltpu.sync_copy(gmax_hbm, gmax_t)

    @pl.when(t == 0)
    def _():
        pltpu.sync_copy(el_hbm, el_s)
        pltpu.sync_copy(er_hbm, er_s)

    zv = jnp.zeros((16,), jnp.float32)

    for v in range(5):
        el_c[0][pl.ds(v * 16, 16)] = zv

    @pl.when(t < 5)
    def _():
        def _zden(i, carry):
            pltpu.sync_copy(el_c[0],
                            den_s.at[pl.ds(t * 2000 + i * CH, CH)])
            return carry
        lax.fori_loop(0, 2000 // CH, _zden, 0)

    def _zrows(i, carry):
        for k in range(8):
            rows_v[0][i, pl.ds(k * 16, 16)] = zv
        return carry
    lax.fori_loop(0, CH, _zrows, 0)

    def _zacc(k, carry):
        pltpu.sync_copy(rows_v[0], acc_s.at[pl.ds(t * TOUT + k * CH, CH)])
        return carry
    lax.fori_loop(0, WOUT // CH, _zacc, 0)

    gmax = gmax_t[...]

    plsc.subcore_barrier()

    def _group(g, carry0):
        base = wid * EB + g * BG
        pltpu.sync_copy(src_hbm.at[pl.ds(base, BG)], src_st)
        pltpu.sync_copy(dst_hbm.at[pl.ds(base, BG)], dst_st)

        def _body(ib, carry):
            for k in range(DEPTH):
                i = ib * DEPTH + k
                for v in range(5):
                    src_sc[k][pl.ds(v * 16, 16)] = (
                        src_st[pl.ds(i * CH + v * 16, 16)])
                    dst_sc[k][pl.ds(v * 16, 16)] = (
                        dst_st[pl.ds(i * CH + v * 16, 16)])
            gds = [None] * DEPTH
            for k in range(NROW):
                gds[k] = pltpu.async_copy(
                    feat_hbm.at[src_sc[k]], rows_v[k], gsem[k])
            eds = []
            for k in range(DEPTH):
                eds.append((
                    pltpu.async_copy(el_s.at[src_sc[k]], el_c[k],
                                     esem[2 * k]),
                    pltpu.async_copy(er_s.at[dst_sc[k]], er_c[k],
                                     esem[2 * k + 1]),
                ))

            sds = [None] * DEPTH
            dds = []
            for k in range(DEPTH):
                eds[k][0].wait()
                eds[k][1].wait()
                for v in range(5):
                    elv = el_c[k][pl.ds(v * 16, 16)]
                    erv = er_c[k][pl.ds(v * 16, 16)]
                    e = _leaky(elv + erv)
                    cv = _leaky(gmax + erv)
                    w_buf[k][pl.ds(v * 16, 16)] = jnp.exp(e - cv)
                dds.append(pltpu.async_copy(
                    w_buf[k], den_s.at[dst_sc[k]], dsem[k], add=True))

                gds[k].wait()
                r = rows_v[k % NROW]

                def _scale(j, carry2, r=r, k=k):
                    aj = plsc.load_gather(
                        w_buf[k], [jnp.full((16,), j, jnp.int32)])
                    for f8 in range(8):
                        r[j, pl.ds(f8 * 16, 16)] = (
                            r[j, pl.ds(f8 * 16, 16)] * aj)
                    return carry2
                lax.fori_loop(0, CH, _scale, 0, unroll=4)

                sds[k] = pltpu.async_copy(
                    r, acc_s.at[dst_sc[k]], ssem[k], add=True)
                if k == DEPTH - NROW:
                    sds[0].wait()
                    gds[DEPTH - 1] = pltpu.async_copy(
                        feat_hbm.at[src_sc[DEPTH - 1]],
                        rows_v[0], gsem[0])
            for k in range(1, DEPTH):
                sds[k].wait()
            for d in dds:
                d.wait()
            return carry
        lax.fori_loop(0, BG // CH // DEPTH, _body, 0)
        return carry0
    lax.fori_loop(0, EB // BG, _group, 0)

    plsc.subcore_barrier()
    pltpu.sync_copy(acc_s.at[pl.ds(t * TOUT, WOUT)],
                    acc_hbm.at[pl.ds(c * N + t * TOUT, WOUT)])
    pltpu.sync_copy(den_s.at[pl.ds(t * TOUT, WOUT)], den_o)
    pltpu.sync_copy(den_o, den_hbm.at[pl.ds(c * N + t * TOUT, WOUT)])


def _sc_edge_wrap(src_hbm, dst_hbm, el_hbm, er_hbm, gmax_hbm, feat_hbm,
                  acc_hbm, den_hbm, gmax_t, src_st, dst_st, *rest):
    src_sc = list(rest[0:DEPTH])
    dst_sc = list(rest[DEPTH:2 * DEPTH])
    w_buf = list(rest[2 * DEPTH:3 * DEPTH])
    el_c = list(rest[3 * DEPTH:4 * DEPTH])
    er_c = list(rest[4 * DEPTH:5 * DEPTH])
    rows_v = list(rest[5 * DEPTH:5 * DEPTH + NROW])
    o = 5 * DEPTH + NROW
    den_o = rest[o]
    o += 1
    el_s, er_s, den_s, acc_s = rest[o:o + 4]
    gsem = list(rest[o + 4:o + 4 + NROW])
    ssem = list(rest[o + 4 + NROW:o + 4 + NROW + DEPTH])
    b = o + 4 + NROW + DEPTH
    dsem = list(rest[b:b + DEPTH])
    esem = list(rest[b + DEPTH:b + 3 * DEPTH])
    _sc_edge_body(src_hbm, dst_hbm, el_hbm, er_hbm, gmax_hbm, feat_hbm,
                  acc_hbm, den_hbm, gmax_t, src_st, dst_st,
                  src_sc, dst_sc, w_buf, el_c, er_c, rows_v, den_o,
                  el_s, er_s, den_s, acc_s, gsem, ssem, dsem, esem)


_sc_edge = functools.partial(
    pl.kernel,
    out_type=(
        jax.ShapeDtypeStruct((NC * N, F), jnp.float32),
        jax.ShapeDtypeStruct((NC * N,), jnp.float32),
    ),
    mesh=plsc.VectorSubcoreMesh(core_axis_name="c", subcore_axis_name="s"),
    compiler_params=pltpu.CompilerParams(needs_layout_passes=False),
    scratch_types=[
        pltpu.VMEM((16,), jnp.float32),
        pltpu.VMEM((BG,), jnp.int32),
        pltpu.VMEM((BG,), jnp.int32),
        *[pltpu.VMEM((CH,), jnp.int32) for _ in range(DEPTH)],
        *[pltpu.VMEM((CH,), jnp.int32) for _ in range(DEPTH)],
        *[pltpu.VMEM((CH,), jnp.float32) for _ in range(DEPTH)],
        *[pltpu.VMEM((CH,), jnp.float32) for _ in range(DEPTH)],
        *[pltpu.VMEM((CH,), jnp.float32) for _ in range(DEPTH)],
        *[pltpu.VMEM((CH, F), jnp.float32) for _ in range(NROW)],
        pltpu.VMEM((WOUT,), jnp.float32),
        pltpu.VMEM_SHARED((N,), jnp.float32),
        pltpu.VMEM_SHARED((N,), jnp.float32),
        pltpu.VMEM_SHARED((N,), jnp.float32),
        pltpu.VMEM_SHARED((N, F), jnp.float32),
        *[pltpu.SemaphoreType.DMA for _ in range(NROW)],
        *[pltpu.SemaphoreType.DMA for _ in range(DEPTH)],
        *[pltpu.SemaphoreType.DMA for _ in range(DEPTH)],
        *[pltpu.SemaphoreType.DMA for _ in range(2 * DEPTH)],
    ],
)(_sc_edge_wrap)


def kernel(x, edge_index, W, attn_l, attn_r, bias):
    src = edge_index[0]
    dst = edge_index[1]

    feat, el2, er2, gm = _tc_proj(x, W,
                                  attn_l.reshape(1, F), attn_r.reshape(1, F))
    el = el2.reshape(N)
    er = er2.reshape(N)
    gm16 = gm.reshape(F)[:16]

    acc, den = _sc_edge(src, dst, el, er, gm16, feat)
    parts = acc.reshape(NC, N, F)
    dens = den.reshape(NC, N, 1)

    out = _tc_combine(parts, dens, x, bias.reshape(1, F))
    return out.reshape(N, H, F)
